# baseline (speedup 1.0000x reference)
import functools

import jax
import jax.numpy as jnp
from jax import lax
from jax.experimental import pallas as pl
from jax.experimental.pallas import tpu as pltpu
from jax.experimental.pallas import tpu_sc as plsc

B = 4096
T = 125
NT = 100000
EMB = 64
ROWS = 127
TP = 128
D_OUT = ROWS * EMB
NW = 32
B_PER_W = B // NW
STEP = 8
NSTEP = B_PER_W // STEP


def _dense_block(tw_ref, w_ref, b_ref, o_ref):
    o_ref[...] = (
        jnp.dot(tw_ref[...], w_ref[...], preferred_element_type=jnp.float32)
        + b_ref[...]
    )


def _compact_block(x_ref, o_ref):
    lane = lax.broadcasted_iota(jnp.int32, (8, 128), 1)
    msk = lane < EMB
    for j in range(63):
        a = x_ref[:, 2 * j, :]
        b = x_ref[:, 2 * j + 1, :]
        o_ref[:, 128 * j : 128 * (j + 1)] = jnp.where(
            msk, a, pltpu.roll(b, EMB, 1)
        )
    o_ref[:, 8064:8128] = x_ref[:, 126, :EMB]


def _tc_compact(x):
    n = x.shape[0]
    return pl.pallas_call(
        _compact_block,
        grid=(n // 8,),
        in_specs=[pl.BlockSpec((8, TP, 128), lambda i: (i, 0, 0))],
        out_specs=pl.BlockSpec((8, D_OUT), lambda i: (i, 0)),
        out_shape=jax.ShapeDtypeStruct((n, D_OUT), jnp.float32),
    )(x)


def _task_dense(task_w, dense_w, dense_b):
    return pl.pallas_call(
        _dense_block,
        grid=(B // 256,),
        in_specs=[
            pl.BlockSpec((256, 16), lambda i: (i, 0)),
            pl.BlockSpec((16, 128), lambda i: (0, 0)),
            pl.BlockSpec((1, 128), lambda i: (0, 0)),
        ],
        out_specs=pl.BlockSpec((256, 128), lambda i: (i, 0)),
        out_shape=jax.ShapeDtypeStruct((B, 128), jnp.float32),
    )(task_w, dense_w, dense_b.reshape(1, 128))


def _sc_gather(idx, table):
    info = plsc.get_sparse_core_info()
    nc = info.num_cores
    mesh = plsc.VectorSubcoreMesh(core_axis_name="c", subcore_axis_name="s")

    @functools.partial(
        pl.kernel,
        mesh=mesh,
        out_type=jax.ShapeDtypeStruct((B, TP, 128), jnp.float32),
        scratch_types=[
            pltpu.VMEM((B_PER_W, TP), jnp.int32),
            pltpu.VMEM((TP, 128), jnp.float32),
            pltpu.VMEM((TP, 128), jnp.float32),
            pltpu.VMEM((TP, 128), jnp.float32),
            pltpu.VMEM((TP, 128), jnp.float32),
            pltpu.SemaphoreType.DMA,
            pltpu.SemaphoreType.DMA,
            pltpu.SemaphoreType.DMA,
            pltpu.SemaphoreType.DMA,
            pltpu.SemaphoreType.DMA,
            pltpu.SemaphoreType.DMA,
            pltpu.SemaphoreType.DMA,
            pltpu.SemaphoreType.DMA,
        ],
    )
    def k(idx_hbm, table_hbm, out_hbm, idx_v, bf0, bf1, bf2, bf3,
          sg0, sg1, sg2, sg3, ss0, ss1, ss2, ss3):
        wid = lax.axis_index("s") * nc + lax.axis_index("c")
        b0 = wid * B_PER_W
        bufs = (bf0, bf1, bf2, bf3)
        sg = (sg0, sg1, sg2, sg3)
        ss = (ss0, ss1, ss2, ss3)
        pltpu.sync_copy(idx_hbm.at[pl.ds(b0, B_PER_W), :], idx_v)

        def gat(i, p):
            pltpu.async_copy(table_hbm.at[idx_v.at[i]], bufs[p], sg[p])

        def wait_gat(p):
            pltpu.make_async_copy(
                table_hbm.at[idx_v.at[0]], bufs[p], sg[p]
            ).wait()

        def sto(i, p):
            pltpu.async_copy(bufs[p], out_hbm.at[b0 + i], ss[p])

        def wait_sto(p):
            pltpu.make_async_copy(bufs[p], out_hbm.at[b0], ss[p]).wait()

        gat(0, 0)
        gat(1, 1)
        for q in range(STEP):
            wait_gat(q % 4)
            sto(q, q % 4)
            if q >= 2:
                wait_sto((q + 2) % 4)
            gat(q + 2, (q + 2) % 4)

        def per_step(s, carry):
            i0 = s * STEP
            for q in range(STEP):
                wait_gat(q % 4)
                sto(i0 + q, q % 4)
                wait_sto((q + 2) % 4)
                gat(i0 + q + 2, (q + 2) % 4)
            return carry

        lax.fori_loop(1, NSTEP - 1, per_step, 0)

        i0 = B_PER_W - STEP
        for q in range(STEP):
            wait_gat(q % 4)
            sto(i0 + q, q % 4)
            if q < STEP - 2:
                wait_sto((q + 2) % 4)
                gat(i0 + q + 2, (q + 2) % 4)
        for p in range(4):
            wait_sto(p)

    return k(idx, table)


def kernel(image, position, direction, prev_action, task_w, embed_table, dense_w, dense_b):
    flat = jnp.concatenate(
        (
            image.reshape(image.shape[0], -1),
            position,
            direction[:, None],
            prev_action[:, None],
        ),
        axis=-1,
    ).astype(jnp.int32)
    task_ids = NT + 2 * jnp.arange(B, dtype=jnp.int32)[:, None] + jnp.arange(
        2, dtype=jnp.int32
    )[None, :]
    idx = jnp.concatenate((flat, task_ids, flat[:, :1]), axis=-1)
    tw = _task_dense(task_w.astype(jnp.float32), dense_w, dense_b)
    table_cat = jnp.concatenate(
        (embed_table, tw.reshape(2 * B, EMB)), axis=0
    )
    tblp = jnp.pad(table_cat, ((0, 0), (0, 128 - EMB)))
    out3 = _sc_gather(idx, tblp)
    return _tc_compact(out3)


# speedup vs baseline: 1.7265x; 1.7265x over previous
import functools

import jax
import jax.numpy as jnp
from jax import lax
from jax.experimental import pallas as pl
from jax.experimental.pallas import tpu as pltpu
from jax.experimental.pallas import tpu_sc as plsc

B = 4096
T = 125
NT = 100000
EMB = 64
ROWS = 127
TP = 128
D_OUT = ROWS * EMB
NW = 32
B_PER_W = B // NW
STEP = 8
NSTEP = B_PER_W // STEP


def _dense_block(tw_ref, w_ref, b_ref, o_ref):
    o_ref[...] = (
        jnp.dot(tw_ref[...], w_ref[...], preferred_element_type=jnp.float32)
        + b_ref[...]
    )


def _compact_block(x_ref, o_ref):
    lane = lax.broadcasted_iota(jnp.int32, (8, 128), 1)
    msk = lane < EMB
    for j in range(63):
        a = x_ref[:, 2 * j, :]
        b = x_ref[:, 2 * j + 1, :]
        o_ref[:, 128 * j : 128 * (j + 1)] = jnp.where(
            msk, a, pltpu.roll(b, EMB, 1)
        )
    o_ref[:, 8064:8128] = x_ref[:, 126, :EMB]


def _tc_compact(x):
    n = x.shape[0]
    return pl.pallas_call(
        _compact_block,
        grid=(n // 8,),
        in_specs=[pl.BlockSpec((8, TP, 128), lambda i: (i, 0, 0))],
        out_specs=pl.BlockSpec((8, D_OUT), lambda i: (i, 0)),
        out_shape=jax.ShapeDtypeStruct((n, D_OUT), jnp.float32),
    )(x)


def _task_dense(task_w, dense_w, dense_b):
    return pl.pallas_call(
        _dense_block,
        grid=(B // 256,),
        in_specs=[
            pl.BlockSpec((256, 16), lambda i: (i, 0)),
            pl.BlockSpec((16, 128), lambda i: (0, 0)),
            pl.BlockSpec((1, 128), lambda i: (0, 0)),
        ],
        out_specs=pl.BlockSpec((256, 128), lambda i: (i, 0)),
        out_shape=jax.ShapeDtypeStruct((B, 128), jnp.float32),
    )(task_w, dense_w, dense_b.reshape(1, 128))


def _sc_gather(idx, table):
    info = plsc.get_sparse_core_info()
    nc = info.num_cores
    mesh = plsc.VectorSubcoreMesh(core_axis_name="c", subcore_axis_name="s")

    @functools.partial(
        pl.kernel,
        mesh=mesh,
        out_type=jax.ShapeDtypeStruct((B, TP, 128), jnp.float32),
        scratch_types=[
            pltpu.VMEM((B_PER_W, TP), jnp.int32),
            pltpu.VMEM((TP, 128), jnp.float32),
            pltpu.VMEM((TP, 128), jnp.float32),
            pltpu.VMEM((TP, 128), jnp.float32),
            pltpu.VMEM((TP, 128), jnp.float32),
            pltpu.SemaphoreType.DMA,
            pltpu.SemaphoreType.DMA,
            pltpu.SemaphoreType.DMA,
            pltpu.SemaphoreType.DMA,
            pltpu.SemaphoreType.DMA,
            pltpu.SemaphoreType.DMA,
            pltpu.SemaphoreType.DMA,
            pltpu.SemaphoreType.DMA,
        ],
    )
    def k(idx_hbm, table_hbm, out_hbm, idx_v, bf0, bf1, bf2, bf3,
          sg0, sg1, sg2, sg3, ss0, ss1, ss2, ss3):
        wid = lax.axis_index("s") * nc + lax.axis_index("c")
        b0 = wid * B_PER_W
        bufs = (bf0, bf1, bf2, bf3)
        sg = (sg0, sg1, sg2, sg3)
        ss = (ss0, ss1, ss2, ss3)
        pltpu.sync_copy(idx_hbm.at[pl.ds(b0, B_PER_W), :], idx_v)

        def gat(i, p):
            pltpu.async_copy(table_hbm.at[idx_v.at[i]], bufs[p], sg[p])

        def wait_gat(p):
            pltpu.make_async_copy(
                table_hbm.at[idx_v.at[0]], bufs[p], sg[p]
            ).wait()

        def sto(i, p):
            pltpu.async_copy(bufs[p], out_hbm.at[b0 + i], ss[p])

        def wait_sto(p):
            pltpu.make_async_copy(bufs[p], out_hbm.at[b0], ss[p]).wait()

        gat(0, 0)
        gat(1, 1)
        gat(2, 2)
        for q in range(STEP):
            wait_gat(q % 4)
            sto(q, q % 4)
            if q >= 1:
                wait_sto((q + 3) % 4)
            gat(q + 3, (q + 3) % 4)

        def per_step(s, carry):
            i0 = s * STEP
            for q in range(STEP):
                wait_gat(q % 4)
                sto(i0 + q, q % 4)
                wait_sto((q + 3) % 4)
                gat(i0 + q + 3, (q + 3) % 4)
            return carry

        lax.fori_loop(1, NSTEP - 1, per_step, 0)

        i0 = B_PER_W - STEP
        for q in range(STEP):
            wait_gat(q % 4)
            sto(i0 + q, q % 4)
            if q < STEP - 3:
                wait_sto((q + 3) % 4)
                gat(i0 + q + 3, (q + 3) % 4)
        for p in range(4):
            wait_sto(p)

    return k(idx, table)


def kernel(image, position, direction, prev_action, task_w, embed_table, dense_w, dense_b):
    flat = jnp.concatenate(
        (
            image.reshape(image.shape[0], -1),
            position,
            direction[:, None],
            prev_action[:, None],
        ),
        axis=-1,
    ).astype(jnp.int32)
    task_ids = NT + 2 * jnp.arange(B, dtype=jnp.int32)[:, None] + jnp.arange(
        2, dtype=jnp.int32
    )[None, :]
    idx = jnp.concatenate((flat, task_ids, flat[:, :1]), axis=-1)
    tw = _task_dense(task_w.astype(jnp.float32), dense_w, dense_b)
    table_cat = jnp.concatenate(
        (embed_table, tw.reshape(2 * B, EMB)), axis=0
    )
    tblp = jnp.pad(table_cat, ((0, 0), (0, 128 - EMB)))
    out3 = _sc_gather(idx, tblp)
    return out3[:, :ROWS, :EMB].reshape(B, D_OUT)
